# baseline (speedup 1.0000x reference)
import functools

import jax
import jax.numpy as jnp
from jax import lax
from jax.experimental import pallas as pl
from jax.experimental.pallas import tpu as pltpu
from jax.experimental.pallas import tpu_sc as plsc

N_NODES = 10000
N_EDGES = 160000
IN_F = 256
HID_F = 1024

CHUNK_W = 128
N_TILES = 16
EDGES_PER_TILE = 10240
EDGE_BLK = 128
N_BLKS = EDGES_PER_TILE // EDGE_BLK
ACC_ROWS = 10240
ROWS_PER_TILE = ACC_ROWS // N_TILES
DRAIN_BLK = 32
N_DRAIN = ROWS_PER_TILE // DRAIN_BLK


def _make_sc_agg(n_chunks):
    n_pass = n_chunks // 2
    mesh = plsc.VectorSubcoreMesh(core_axis_name="c", subcore_axis_name="s")

    @functools.partial(
        pl.kernel,
        out_type=jax.ShapeDtypeStruct((n_chunks, ACC_ROWS, CHUNK_W), jnp.float32),
        mesh=mesh,
        scratch_types=[
            pltpu.VMEM_SHARED((ACC_ROWS, CHUNK_W), jnp.float32),
            pltpu.VMEM((EDGE_BLK,), jnp.int32),
            pltpu.VMEM((EDGE_BLK,), jnp.int32),
            pltpu.VMEM((EDGE_BLK,), jnp.int32),
            pltpu.VMEM((EDGE_BLK,), jnp.int32),
            pltpu.VMEM((EDGE_BLK, CHUNK_W), jnp.float32),
            pltpu.VMEM((EDGE_BLK, CHUNK_W), jnp.float32),
            pltpu.VMEM((DRAIN_BLK, CHUNK_W), jnp.float32),
            pltpu.SemaphoreType.DMA,
            pltpu.SemaphoreType.DMA,
            pltpu.SemaphoreType.DMA,
            pltpu.SemaphoreType.DMA,
            pltpu.SemaphoreType.DMA,
            pltpu.SemaphoreType.DMA,
            pltpu.SemaphoreType.DMA,
        ],
    )
    def sc_agg(*refs):
        xs = refs[:n_chunks]
        src_hbm, dst_hbm, out_hbm = refs[n_chunks:n_chunks + 3]
        (agg_sh, s0, s1, d0, d1, rows0, rows1, zd_v,
         gsem0, gsem1, dsem0, dsem1, ssem0, ssem1, zsem) = refs[n_chunks + 3:]
        c = lax.axis_index("c")
        s = lax.axis_index("s")
        sbuf = (s0, s1)
        dbuf = (d0, d1)
        rbuf = (rows0, rows1)
        gsem = (gsem0, gsem1)
        dsem = (dsem0, dsem1)
        ssem = (ssem0, ssem1)

        def _zi(i, _):
            for k in range(CHUNK_W // 16):
                zd_v[i, pl.ds(k * 16, 16)] = jnp.zeros((16,), jnp.float32)
            return 0
        lax.fori_loop(0, DRAIN_BLK, _zi, 0)

        def start_b(b, k):
            base = s * EDGES_PER_TILE + b * EDGE_BLK
            pltpu.async_copy(dst_hbm.at[pl.ds(base, EDGE_BLK)], dbuf[k], dsem[k])
            pltpu.async_copy(src_hbm.at[pl.ds(base, EDGE_BLK)], sbuf[k], gsem[k])

        def one_pass(x_hbm, chunk):
            for j in range(N_DRAIN):
                pltpu.async_copy(
                    zd_v,
                    agg_sh.at[pl.ds(s * ROWS_PER_TILE + j * DRAIN_BLK, DRAIN_BLK)],
                    zsem)
            for j in range(N_DRAIN):
                pltpu.make_async_copy(
                    zd_v,
                    agg_sh.at[pl.ds(s * ROWS_PER_TILE + j * DRAIN_BLK, DRAIN_BLK)],
                    zsem).wait()
            plsc.subcore_barrier()

            def fin_idx(b, k):
                base = s * EDGES_PER_TILE + b * EDGE_BLK
                pltpu.make_async_copy(dst_hbm.at[pl.ds(base, EDGE_BLK)],
                                      dbuf[k], dsem[k]).wait()
                pltpu.make_async_copy(src_hbm.at[pl.ds(base, EDGE_BLK)],
                                      sbuf[k], gsem[k]).wait()
                pltpu.async_copy(x_hbm.at[sbuf[k]], rbuf[k], gsem[k])

            start_b(0, 0)
            fin_idx(0, 0)
            start_b(1, 1)
            fin_idx(1, 1)

            def pair(j, _):
                b0 = 2 * j
                pltpu.make_async_copy(x_hbm.at[sbuf[0]], rbuf[0], gsem[0]).wait()
                sc0 = pltpu.async_copy(rbuf[0], agg_sh.at[dbuf[0]], ssem[0],
                                       add=True)
                pltpu.make_async_copy(x_hbm.at[sbuf[1]], rbuf[1], gsem[1]).wait()
                sc1 = pltpu.async_copy(rbuf[1], agg_sh.at[dbuf[1]], ssem[1],
                                       add=True)
                sc0.wait()
                start_b(b0 + 2, 0)
                fin_idx(b0 + 2, 0)
                sc1.wait()
                start_b(b0 + 3, 1)
                fin_idx(b0 + 3, 1)
                return 0
            lax.fori_loop(0, N_BLKS // 2 - 1, pair, 0)
            pltpu.make_async_copy(x_hbm.at[sbuf[0]], rbuf[0], gsem[0]).wait()
            pltpu.sync_copy(rbuf[0], agg_sh.at[dbuf[0]], add=True)
            pltpu.make_async_copy(x_hbm.at[sbuf[1]], rbuf[1], gsem[1]).wait()
            pltpu.sync_copy(rbuf[1], agg_sh.at[dbuf[1]], add=True)
            plsc.subcore_barrier()

            n_dr = ROWS_PER_TILE // EDGE_BLK
            out_desc = [None] * n_dr
            for j in range(n_dr):
                k = j % 2
                if j >= 2:
                    out_desc[j - 2].wait()
                row0 = s * ROWS_PER_TILE + j * EDGE_BLK
                pltpu.async_copy(agg_sh.at[pl.ds(row0, EDGE_BLK)],
                                 rbuf[k], gsem[k]).wait()
                out_desc[j] = pltpu.async_copy(
                    rbuf[k], out_hbm.at[chunk, pl.ds(row0, EDGE_BLK)], ssem[k])
            out_desc[n_dr - 2].wait()
            out_desc[n_dr - 1].wait()

        for p in range(n_pass):
            for ccode in range(2):
                @pl.when(c == ccode)
                def _(p=p, ccode=ccode):
                    one_pass(xs[2 * p + ccode], 2 * p + ccode)

    return sc_agg


_sc_agg_2 = _make_sc_agg(2)
_sc_agg_8 = _make_sc_agg(8)


def _mlp1_body(x_ref, agg_ref, wa_ref, ba_ref, wb_ref, bb_ref, h_ref):
    xin = x_ref[...] + jnp.concatenate([agg_ref[0], agg_ref[1]], axis=-1)
    t = jnp.dot(xin, wa_ref[...], preferred_element_type=jnp.float32) + ba_ref[...]
    t = jnp.maximum(t, 0.0)
    h = jnp.dot(t, wb_ref[...], preferred_element_type=jnp.float32) + bb_ref[...]
    h_ref[...] = jnp.maximum(h, 0.0)


def _mlp2_body(h_ref, agg_ref, wa_ref, ba_ref, wb_ref, bb_ref, o_ref):
    i = pl.program_id(0)
    zin = h_ref[...] + jnp.concatenate(
        [agg_ref[j] for j in range(HID_F // CHUNK_W)], axis=-1)
    t = jnp.dot(zin, wa_ref[...], preferred_element_type=jnp.float32) + ba_ref[...]
    t = jnp.maximum(t, 0.0)
    r = jnp.dot(t, wb_ref[...], preferred_element_type=jnp.float32) + bb_ref[...]
    r = jnp.maximum(r, 0.0)
    part = jnp.sum(r, axis=0, keepdims=True)

    @pl.when(i == 0)
    def _():
        o_ref[...] = part

    @pl.when(i != 0)
    def _():
        o_ref[...] = o_ref[...] + part


ROW_BLK = 1000
N_ROW_BLKS = N_NODES // ROW_BLK


def _mlp1(x, agg1, W1a, b1a, W1b, b1b):
    return pl.pallas_call(
        _mlp1_body,
        grid=(N_ROW_BLKS,),
        in_specs=[
            pl.BlockSpec((ROW_BLK, IN_F), lambda i: (i, 0)),
            pl.BlockSpec((2, ROW_BLK, CHUNK_W), lambda i: (0, i, 0)),
            pl.BlockSpec((IN_F, HID_F), lambda i: (0, 0)),
            pl.BlockSpec((1, HID_F), lambda i: (0, 0)),
            pl.BlockSpec((HID_F, HID_F), lambda i: (0, 0)),
            pl.BlockSpec((1, HID_F), lambda i: (0, 0)),
        ],
        out_specs=pl.BlockSpec((ROW_BLK, HID_F), lambda i: (i, 0)),
        out_shape=jax.ShapeDtypeStruct((N_NODES, HID_F), jnp.float32),
    )(x, agg1, W1a, b1a.reshape(1, -1), W1b, b1b.reshape(1, -1))


def _mlp2(h, agg2, W2a, b2a, W2b, b2b):
    out = pl.pallas_call(
        _mlp2_body,
        grid=(N_ROW_BLKS,),
        in_specs=[
            pl.BlockSpec((ROW_BLK, HID_F), lambda i: (i, 0)),
            pl.BlockSpec((HID_F // CHUNK_W, ROW_BLK, CHUNK_W), lambda i: (0, i, 0)),
            pl.BlockSpec((HID_F, HID_F), lambda i: (0, 0)),
            pl.BlockSpec((1, HID_F), lambda i: (0, 0)),
            pl.BlockSpec((HID_F, IN_F), lambda i: (0, 0)),
            pl.BlockSpec((1, IN_F), lambda i: (0, 0)),
        ],
        out_specs=pl.BlockSpec((1, IN_F), lambda i: (0, 0)),
        out_shape=jax.ShapeDtypeStruct((1, IN_F), jnp.float32),
    )(h, agg2, W2a, b2a.reshape(1, -1), W2b, b2b.reshape(1, -1))
    return out.reshape(IN_F)


def kernel(x, edge_index, W1a, b1a, W1b, b1b, W2a, b2a, W2b, b2b):
    e = edge_index.astype(jnp.int32)
    pad = EDGES_PER_TILE - N_EDGES // N_TILES
    src = jnp.pad(e[0].reshape(N_TILES, -1), ((0, 0), (0, pad)),
                  constant_values=0).reshape(-1)
    dst = jnp.pad(e[1].reshape(N_TILES, -1), ((0, 0), (0, pad)),
                  constant_values=ACC_ROWS - 1).reshape(-1)

    x_chunks = tuple(x[:, i * CHUNK_W:(i + 1) * CHUNK_W]
                     for i in range(IN_F // CHUNK_W))
    agg1 = _sc_agg_2(*x_chunks, src, dst)[:, :N_NODES]

    h = _mlp1(x, agg1, W1a, b1a, W1b, b1b)

    h_chunks = tuple(h[:, i * CHUNK_W:(i + 1) * CHUNK_W]
                     for i in range(HID_F // CHUNK_W))
    agg2 = _sc_agg_8(*h_chunks, src, dst)[:, :N_NODES]

    return _mlp2(h, agg2, W2a, b2a, W2b, b2b)


# speedup vs baseline: 1.4366x; 1.4366x over previous
import functools

import jax
import jax.numpy as jnp
from jax import lax
from jax.experimental import pallas as pl
from jax.experimental.pallas import tpu as pltpu
from jax.experimental.pallas import tpu_sc as plsc

N_NODES = 10000
N_EDGES = 160000
IN_F = 256
HID_F = 1024

SL = 2
LN = 128
CHUNK_W = SL * LN
N_TILES = 16
EDGES_PER_TILE = 10240
EDGE_BLK = 128
N_BLKS = EDGES_PER_TILE // EDGE_BLK
ACC_ROWS = 10240
ROWS_PER_TILE = ACC_ROWS // N_TILES
ZERO_BLK = 32


def _make_sc_agg(n_chunks):
    n_out = 2 if n_chunks == 1 else n_chunks
    mesh = plsc.VectorSubcoreMesh(core_axis_name="c", subcore_axis_name="s")

    @functools.partial(
        pl.kernel,
        out_type=jax.ShapeDtypeStruct((n_out, ACC_ROWS, SL, LN), jnp.bfloat16),
        mesh=mesh,
        compiler_params=pltpu.CompilerParams(use_tc_tiling_on_sc=False),
        scratch_types=[
            pltpu.VMEM_SHARED((ACC_ROWS, SL, LN), jnp.bfloat16),
            pltpu.VMEM((EDGE_BLK,), jnp.int32),
            pltpu.VMEM((EDGE_BLK,), jnp.int32),
            pltpu.VMEM((EDGE_BLK,), jnp.int32),
            pltpu.VMEM((EDGE_BLK,), jnp.int32),
            pltpu.VMEM((EDGE_BLK, SL, LN), jnp.bfloat16),
            pltpu.VMEM((EDGE_BLK, SL, LN), jnp.bfloat16),
            pltpu.VMEM((ZERO_BLK, SL, LN), jnp.bfloat16),
            pltpu.SemaphoreType.DMA,
            pltpu.SemaphoreType.DMA,
            pltpu.SemaphoreType.DMA,
            pltpu.SemaphoreType.DMA,
            pltpu.SemaphoreType.DMA,
            pltpu.SemaphoreType.DMA,
            pltpu.SemaphoreType.DMA,
        ],
    )
    def sc_agg(*refs):
        xs = refs[:n_chunks]
        src_hbm, dst_hbm, out_hbm = refs[n_chunks:n_chunks + 3]
        (agg_sh, s0, s1, d0, d1, rows0, rows1, zd_v,
         gsem0, gsem1, dsem0, dsem1, ssem0, ssem1, zsem) = refs[n_chunks + 3:]
        c = lax.axis_index("c")
        s = lax.axis_index("s")
        sbuf = (s0, s1)
        dbuf = (d0, d1)
        rbuf = (rows0, rows1)
        gsem = (gsem0, gsem1)
        dsem = (dsem0, dsem1)
        ssem = (ssem0, ssem1)

        def _zi(i, _):
            for t in range(SL):
                for k in range(LN // 32):
                    zd_v[i, t, pl.ds(k * 32, 32)] = jnp.zeros((32,), jnp.bfloat16)
            return 0
        lax.fori_loop(0, ZERO_BLK, _zi, 0)

        def one_pass(x_hbm, oi, blk0, nblk):
            nz = ROWS_PER_TILE // ZERO_BLK
            for j in range(nz):
                pltpu.async_copy(
                    zd_v,
                    agg_sh.at[pl.ds(s * ROWS_PER_TILE + j * ZERO_BLK, ZERO_BLK)],
                    zsem)
            for j in range(nz):
                pltpu.make_async_copy(
                    zd_v,
                    agg_sh.at[pl.ds(s * ROWS_PER_TILE + j * ZERO_BLK, ZERO_BLK)],
                    zsem).wait()
            plsc.subcore_barrier()

            def start_b(b, k):
                base = s * EDGES_PER_TILE + (blk0 + b) * EDGE_BLK
                pltpu.async_copy(dst_hbm.at[pl.ds(base, EDGE_BLK)], dbuf[k], dsem[k])
                pltpu.async_copy(src_hbm.at[pl.ds(base, EDGE_BLK)], sbuf[k], gsem[k])

            def fin_idx(b, k):
                base = s * EDGES_PER_TILE + (blk0 + b) * EDGE_BLK
                pltpu.make_async_copy(dst_hbm.at[pl.ds(base, EDGE_BLK)],
                                      dbuf[k], dsem[k]).wait()
                pltpu.make_async_copy(src_hbm.at[pl.ds(base, EDGE_BLK)],
                                      sbuf[k], gsem[k]).wait()
                pltpu.async_copy(x_hbm.at[sbuf[k]], rbuf[k], gsem[k])

            start_b(0, 0)
            fin_idx(0, 0)
            start_b(1, 1)
            fin_idx(1, 1)

            def pair(j, _):
                b0 = 2 * j
                pltpu.make_async_copy(x_hbm.at[sbuf[0]], rbuf[0], gsem[0]).wait()
                sc0 = pltpu.async_copy(rbuf[0], agg_sh.at[dbuf[0]], ssem[0],
                                       add=True)
                pltpu.make_async_copy(x_hbm.at[sbuf[1]], rbuf[1], gsem[1]).wait()
                sc1 = pltpu.async_copy(rbuf[1], agg_sh.at[dbuf[1]], ssem[1],
                                       add=True)
                sc0.wait()
                start_b(b0 + 2, 0)
                fin_idx(b0 + 2, 0)
                sc1.wait()
                start_b(b0 + 3, 1)
                fin_idx(b0 + 3, 1)
                return 0
            lax.fori_loop(0, nblk // 2 - 1, pair, 0)
            pltpu.make_async_copy(x_hbm.at[sbuf[0]], rbuf[0], gsem[0]).wait()
            pltpu.sync_copy(rbuf[0], agg_sh.at[dbuf[0]], add=True)
            pltpu.make_async_copy(x_hbm.at[sbuf[1]], rbuf[1], gsem[1]).wait()
            pltpu.sync_copy(rbuf[1], agg_sh.at[dbuf[1]], add=True)
            plsc.subcore_barrier()

            n_dr = ROWS_PER_TILE // EDGE_BLK
            out_desc = [None] * n_dr
            for j in range(n_dr):
                k = j % 2
                if j >= 2:
                    out_desc[j - 2].wait()
                row0 = s * ROWS_PER_TILE + j * EDGE_BLK
                pltpu.async_copy(agg_sh.at[pl.ds(row0, EDGE_BLK)],
                                 rbuf[k], gsem[k]).wait()
                out_desc[j] = pltpu.async_copy(
                    rbuf[k], out_hbm.at[oi, pl.ds(row0, EDGE_BLK)], ssem[k])
            out_desc[n_dr - 2].wait()
            out_desc[n_dr - 1].wait()

        if n_chunks == 1:
            for ccode in range(2):
                @pl.when(c == ccode)
                def _(ccode=ccode):
                    one_pass(xs[0], ccode, ccode * (N_BLKS // 2), N_BLKS // 2)
        else:
            for p in range(n_chunks // 2):
                for ccode in range(2):
                    @pl.when(c == ccode)
                    def _(p=p, ccode=ccode):
                        one_pass(xs[2 * p + ccode], 2 * p + ccode, 0, N_BLKS)

    return sc_agg


_sc_agg_1 = _make_sc_agg(1)
_sc_agg_4 = _make_sc_agg(4)


def _mlp1_body(x_ref, agg_ref, wa_ref, ba_ref, wb_ref, bb_ref, h_ref):
    agg = (agg_ref[0].astype(jnp.float32) + agg_ref[1].astype(jnp.float32))
    xin = x_ref[...] + agg
    t = jnp.dot(xin, wa_ref[...], preferred_element_type=jnp.float32) + ba_ref[...]
    t = jnp.maximum(t, 0.0)
    h = jnp.dot(t, wb_ref[...], preferred_element_type=jnp.float32) + bb_ref[...]
    h_ref[...] = jnp.maximum(h, 0.0)


def _mlp2_body(h_ref, agg_ref, wa_ref, ba_ref, wb_ref, bb_ref, o_ref):
    i = pl.program_id(0)
    zin = h_ref[...] + jnp.concatenate(
        [agg_ref[j].astype(jnp.float32) for j in range(HID_F // CHUNK_W)], axis=-1)
    t = jnp.dot(zin, wa_ref[...], preferred_element_type=jnp.float32) + ba_ref[...]
    t = jnp.maximum(t, 0.0)
    r = jnp.dot(t, wb_ref[...], preferred_element_type=jnp.float32) + bb_ref[...]
    r = jnp.maximum(r, 0.0)
    part = jnp.sum(r, axis=0, keepdims=True)

    @pl.when(i == 0)
    def _():
        o_ref[...] = part

    @pl.when(i != 0)
    def _():
        o_ref[...] = o_ref[...] + part


ROW_BLK = 1000
N_ROW_BLKS = N_NODES // ROW_BLK


def _mlp1(x, agg1, W1a, b1a, W1b, b1b):
    return pl.pallas_call(
        _mlp1_body,
        grid=(N_ROW_BLKS,),
        in_specs=[
            pl.BlockSpec((ROW_BLK, IN_F), lambda i: (i, 0)),
            pl.BlockSpec((2, ROW_BLK, CHUNK_W), lambda i: (0, i, 0)),
            pl.BlockSpec((IN_F, HID_F), lambda i: (0, 0)),
            pl.BlockSpec((1, HID_F), lambda i: (0, 0)),
            pl.BlockSpec((HID_F, HID_F), lambda i: (0, 0)),
            pl.BlockSpec((1, HID_F), lambda i: (0, 0)),
        ],
        out_specs=pl.BlockSpec((ROW_BLK, HID_F), lambda i: (i, 0)),
        out_shape=jax.ShapeDtypeStruct((N_NODES, HID_F), jnp.float32),
    )(x, agg1, W1a, b1a.reshape(1, -1), W1b, b1b.reshape(1, -1))


def _mlp2(h, agg2, W2a, b2a, W2b, b2b):
    out = pl.pallas_call(
        _mlp2_body,
        grid=(N_ROW_BLKS,),
        in_specs=[
            pl.BlockSpec((ROW_BLK, HID_F), lambda i: (i, 0)),
            pl.BlockSpec((HID_F // CHUNK_W, ROW_BLK, CHUNK_W), lambda i: (0, i, 0)),
            pl.BlockSpec((HID_F, HID_F), lambda i: (0, 0)),
            pl.BlockSpec((1, HID_F), lambda i: (0, 0)),
            pl.BlockSpec((HID_F, IN_F), lambda i: (0, 0)),
            pl.BlockSpec((1, IN_F), lambda i: (0, 0)),
        ],
        out_specs=pl.BlockSpec((1, IN_F), lambda i: (0, 0)),
        out_shape=jax.ShapeDtypeStruct((1, IN_F), jnp.float32),
    )(h, agg2, W2a, b2a.reshape(1, -1), W2b, b2b.reshape(1, -1))
    return out.reshape(IN_F)


def kernel(x, edge_index, W1a, b1a, W1b, b1b, W2a, b2a, W2b, b2b):
    e = edge_index.astype(jnp.int32)
    pad = EDGES_PER_TILE - N_EDGES // N_TILES
    src = jnp.pad(e[0].reshape(N_TILES, -1), ((0, 0), (0, pad)),
                  constant_values=0).reshape(-1)
    dst = jnp.pad(e[1].reshape(N_TILES, -1), ((0, 0), (0, pad)),
                  constant_values=ACC_ROWS - 1).reshape(-1)

    xb = x.astype(jnp.bfloat16).reshape(N_NODES, SL, LN)
    agg1 = _sc_agg_1(xb, src, dst)
    agg1 = agg1.reshape(2, ACC_ROWS, CHUNK_W)[:, :N_NODES]

    h = _mlp1(x, agg1, W1a, b1a, W1b, b1b)

    hb = h.astype(jnp.bfloat16)
    h_chunks = tuple(
        hb[:, i * CHUNK_W:(i + 1) * CHUNK_W].reshape(N_NODES, SL, LN)
        for i in range(HID_F // CHUNK_W))
    agg2 = _sc_agg_4(*h_chunks, src, dst)
    agg2 = agg2.reshape(4, ACC_ROWS, CHUNK_W)[:, :N_NODES]

    return _mlp2(h, agg2, W2a, b2a, W2b, b2b)


# speedup vs baseline: 1.4412x; 1.0032x over previous
import functools

import jax
import jax.numpy as jnp
from jax import lax
from jax.experimental import pallas as pl
from jax.experimental.pallas import tpu as pltpu
from jax.experimental.pallas import tpu_sc as plsc

N_NODES = 10000
N_EDGES = 160000
IN_F = 256
HID_F = 1024

SL = 2
LN = 128
CHUNK_W = SL * LN
N_TILES = 16
EDGES_PER_TILE = 10240
EDGE_BLK = 128
N_BLKS = EDGES_PER_TILE // EDGE_BLK
ACC_ROWS = 10240
ROWS_PER_TILE = ACC_ROWS // N_TILES
ZERO_BLK = 32


def _make_sc_agg(n_chunks):
    n_out = 2 if n_chunks == 1 else n_chunks
    mesh = plsc.VectorSubcoreMesh(core_axis_name="c", subcore_axis_name="s")

    @functools.partial(
        pl.kernel,
        out_type=jax.ShapeDtypeStruct((n_out, ACC_ROWS, SL, LN), jnp.bfloat16),
        mesh=mesh,
        compiler_params=pltpu.CompilerParams(use_tc_tiling_on_sc=False),
        scratch_types=[
            pltpu.VMEM_SHARED((ACC_ROWS, SL, LN), jnp.bfloat16),
            pltpu.VMEM((EDGE_BLK,), jnp.int32),
            pltpu.VMEM((EDGE_BLK,), jnp.int32),
            pltpu.VMEM((EDGE_BLK,), jnp.int32),
            pltpu.VMEM((EDGE_BLK,), jnp.int32),
            pltpu.VMEM((EDGE_BLK, SL, LN), jnp.bfloat16),
            pltpu.VMEM((EDGE_BLK, SL, LN), jnp.bfloat16),
            pltpu.VMEM((ZERO_BLK, SL, LN), jnp.bfloat16),
            pltpu.SemaphoreType.DMA,
            pltpu.SemaphoreType.DMA,
            pltpu.SemaphoreType.DMA,
            pltpu.SemaphoreType.DMA,
            pltpu.SemaphoreType.DMA,
            pltpu.SemaphoreType.DMA,
            pltpu.SemaphoreType.DMA,
        ],
    )
    def sc_agg(*refs):
        xs = refs[:n_chunks]
        src_hbm, dst_hbm, out_hbm = refs[n_chunks:n_chunks + 3]
        (agg_sh, s0, s1, d0, d1, rows0, rows1, zd_v,
         gsem0, gsem1, dsem0, dsem1, ssem0, ssem1, zsem) = refs[n_chunks + 3:]
        c = lax.axis_index("c")
        s = lax.axis_index("s")
        sbuf = (s0, s1)
        dbuf = (d0, d1)
        rbuf = (rows0, rows1)
        gsem = (gsem0, gsem1)
        dsem = (dsem0, dsem1)
        ssem = (ssem0, ssem1)

        def _zi(i, _):
            for t in range(SL):
                for k in range(LN // 32):
                    zd_v[i, t, pl.ds(k * 32, 32)] = jnp.zeros((32,), jnp.bfloat16)
            return 0
        lax.fori_loop(0, ZERO_BLK, _zi, 0)

        def one_pass(x_hbm, oi, blk0, nblk):
            nz = ROWS_PER_TILE // ZERO_BLK
            for j in range(nz):
                pltpu.async_copy(
                    zd_v,
                    agg_sh.at[pl.ds(s * ROWS_PER_TILE + j * ZERO_BLK, ZERO_BLK)],
                    zsem)
            for j in range(nz):
                pltpu.make_async_copy(
                    zd_v,
                    agg_sh.at[pl.ds(s * ROWS_PER_TILE + j * ZERO_BLK, ZERO_BLK)],
                    zsem).wait()
            plsc.subcore_barrier()

            def start_b(b, k):
                base = s * EDGES_PER_TILE + (blk0 + b) * EDGE_BLK
                pltpu.async_copy(dst_hbm.at[pl.ds(base, EDGE_BLK)], dbuf[k], dsem[k])
                pltpu.async_copy(src_hbm.at[pl.ds(base, EDGE_BLK)], sbuf[k], gsem[k])

            def fin_idx(b, k):
                base = s * EDGES_PER_TILE + (blk0 + b) * EDGE_BLK
                pltpu.make_async_copy(dst_hbm.at[pl.ds(base, EDGE_BLK)],
                                      dbuf[k], dsem[k]).wait()
                pltpu.make_async_copy(src_hbm.at[pl.ds(base, EDGE_BLK)],
                                      sbuf[k], gsem[k]).wait()
                pltpu.async_copy(x_hbm.at[sbuf[k]], rbuf[k], gsem[k])

            start_b(0, 0)
            fin_idx(0, 0)
            start_b(1, 1)
            fin_idx(1, 1)

            def pair(j, _):
                b0 = 2 * j
                pltpu.make_async_copy(x_hbm.at[sbuf[0]], rbuf[0], gsem[0]).wait()
                sc0 = pltpu.async_copy(rbuf[0], agg_sh.at[dbuf[0]], ssem[0],
                                       add=True)
                pltpu.make_async_copy(x_hbm.at[sbuf[1]], rbuf[1], gsem[1]).wait()
                sc1 = pltpu.async_copy(rbuf[1], agg_sh.at[dbuf[1]], ssem[1],
                                       add=True)
                sc0.wait()
                start_b(b0 + 2, 0)
                fin_idx(b0 + 2, 0)
                sc1.wait()
                start_b(b0 + 3, 1)
                fin_idx(b0 + 3, 1)
                return 0
            lax.fori_loop(0, nblk // 2 - 1, pair, 0)
            pltpu.make_async_copy(x_hbm.at[sbuf[0]], rbuf[0], gsem[0]).wait()
            pltpu.sync_copy(rbuf[0], agg_sh.at[dbuf[0]], add=True)
            pltpu.make_async_copy(x_hbm.at[sbuf[1]], rbuf[1], gsem[1]).wait()
            pltpu.sync_copy(rbuf[1], agg_sh.at[dbuf[1]], add=True)
            plsc.subcore_barrier()

            n_dr = ROWS_PER_TILE // EDGE_BLK
            out_desc = [None] * n_dr
            for j in range(n_dr):
                k = j % 2
                if j >= 2:
                    out_desc[j - 2].wait()
                row0 = s * ROWS_PER_TILE + j * EDGE_BLK
                pltpu.async_copy(agg_sh.at[pl.ds(row0, EDGE_BLK)],
                                 rbuf[k], gsem[k]).wait()
                out_desc[j] = pltpu.async_copy(
                    rbuf[k], out_hbm.at[oi, pl.ds(row0, EDGE_BLK)], ssem[k])
            out_desc[n_dr - 2].wait()
            out_desc[n_dr - 1].wait()

        if n_chunks == 1:
            for ccode in range(2):
                @pl.when(c == ccode)
                def _(ccode=ccode):
                    one_pass(xs[0], ccode, ccode * (N_BLKS // 2), N_BLKS // 2)
        else:
            for p in range(n_chunks // 2):
                for ccode in range(2):
                    @pl.when(c == ccode)
                    def _(p=p, ccode=ccode):
                        one_pass(xs[2 * p + ccode], 2 * p + ccode, 0, N_BLKS)

    return sc_agg


_sc_agg_1 = _make_sc_agg(1)
_sc_agg_4 = _make_sc_agg(4)


def _mlp1_body(x_ref, agg_ref, wa_ref, ba_ref, wb_ref, bb_ref, h_ref):
    agg = (agg_ref[0].astype(jnp.float32) + agg_ref[1].astype(jnp.float32))
    xin = (x_ref[...] + agg).astype(jnp.bfloat16)
    t = jnp.dot(xin, wa_ref[...], preferred_element_type=jnp.float32) + ba_ref[...]
    t = jnp.maximum(t, 0.0).astype(jnp.bfloat16)
    h = jnp.dot(t, wb_ref[...], preferred_element_type=jnp.float32) + bb_ref[...]
    h_ref[...] = jnp.maximum(h, 0.0)


def _mlp2_body(h_ref, agg_ref, wa_ref, ba_ref, wb_ref, bb_ref, o_ref):
    i = pl.program_id(0)
    zin = h_ref[...] + jnp.concatenate(
        [agg_ref[j].astype(jnp.float32) for j in range(HID_F // CHUNK_W)], axis=-1)
    zin = zin.astype(jnp.bfloat16)
    t = jnp.dot(zin, wa_ref[...], preferred_element_type=jnp.float32) + ba_ref[...]
    t = jnp.maximum(t, 0.0).astype(jnp.bfloat16)
    r = jnp.dot(t, wb_ref[...], preferred_element_type=jnp.float32) + bb_ref[...]
    r = jnp.maximum(r, 0.0)
    part = jnp.sum(r, axis=0, keepdims=True)

    @pl.when(i == 0)
    def _():
        o_ref[...] = part

    @pl.when(i != 0)
    def _():
        o_ref[...] = o_ref[...] + part


ROW_BLK = 1000
N_ROW_BLKS = N_NODES // ROW_BLK


def _mlp1(x, agg1, W1a, b1a, W1b, b1b):
    return pl.pallas_call(
        _mlp1_body,
        grid=(N_ROW_BLKS,),
        in_specs=[
            pl.BlockSpec((ROW_BLK, IN_F), lambda i: (i, 0)),
            pl.BlockSpec((2, ROW_BLK, CHUNK_W), lambda i: (0, i, 0)),
            pl.BlockSpec((IN_F, HID_F), lambda i: (0, 0)),
            pl.BlockSpec((1, HID_F), lambda i: (0, 0)),
            pl.BlockSpec((HID_F, HID_F), lambda i: (0, 0)),
            pl.BlockSpec((1, HID_F), lambda i: (0, 0)),
        ],
        out_specs=pl.BlockSpec((ROW_BLK, HID_F), lambda i: (i, 0)),
        out_shape=jax.ShapeDtypeStruct((N_NODES, HID_F), jnp.float32),
    )(x, agg1, W1a.astype(jnp.bfloat16), b1a.reshape(1, -1),
      W1b.astype(jnp.bfloat16), b1b.reshape(1, -1))


def _mlp2(h, agg2, W2a, b2a, W2b, b2b):
    out = pl.pallas_call(
        _mlp2_body,
        grid=(N_ROW_BLKS,),
        in_specs=[
            pl.BlockSpec((ROW_BLK, HID_F), lambda i: (i, 0)),
            pl.BlockSpec((HID_F // CHUNK_W, ROW_BLK, CHUNK_W), lambda i: (0, i, 0)),
            pl.BlockSpec((HID_F, HID_F), lambda i: (0, 0)),
            pl.BlockSpec((1, HID_F), lambda i: (0, 0)),
            pl.BlockSpec((HID_F, IN_F), lambda i: (0, 0)),
            pl.BlockSpec((1, IN_F), lambda i: (0, 0)),
        ],
        out_specs=pl.BlockSpec((1, IN_F), lambda i: (0, 0)),
        out_shape=jax.ShapeDtypeStruct((1, IN_F), jnp.float32),
    )(h, agg2, W2a.astype(jnp.bfloat16), b2a.reshape(1, -1),
      W2b.astype(jnp.bfloat16), b2b.reshape(1, -1))
    return out.reshape(IN_F)


def kernel(x, edge_index, W1a, b1a, W1b, b1b, W2a, b2a, W2b, b2b):
    e = edge_index.astype(jnp.int32)
    pad = EDGES_PER_TILE - N_EDGES // N_TILES
    src = jnp.pad(e[0].reshape(N_TILES, -1), ((0, 0), (0, pad)),
                  constant_values=0).reshape(-1)
    dst = jnp.pad(e[1].reshape(N_TILES, -1), ((0, 0), (0, pad)),
                  constant_values=ACC_ROWS - 1).reshape(-1)

    xb = x.astype(jnp.bfloat16).reshape(N_NODES, SL, LN)
    agg1 = _sc_agg_1(xb, src, dst)
    agg1 = agg1.reshape(2, ACC_ROWS, CHUNK_W)[:, :N_NODES]

    h = _mlp1(x, agg1, W1a, b1a, W1b, b1b)

    hb = h.astype(jnp.bfloat16)
    h_chunks = tuple(
        hb[:, i * CHUNK_W:(i + 1) * CHUNK_W].reshape(N_NODES, SL, LN)
        for i in range(HID_F // CHUNK_W))
    agg2 = _sc_agg_4(*h_chunks, src, dst)
    agg2 = agg2.reshape(4, ACC_ROWS, CHUNK_W)[:, :N_NODES]

    return _mlp2(h, agg2, W2a, b2a, W2b, b2b)


# speedup vs baseline: 1.4767x; 1.0246x over previous
import functools

import jax
import jax.numpy as jnp
from jax import lax
from jax.experimental import pallas as pl
from jax.experimental.pallas import tpu as pltpu
from jax.experimental.pallas import tpu_sc as plsc

N_NODES = 10000
N_EDGES = 160000
IN_F = 256
HID_F = 1024

SL = 2
LN = 128
CHUNK_W = SL * LN
N_TILES = 16
EDGES_PER_TILE = 10240
EDGE_BLK = 128
N_BLKS = EDGES_PER_TILE // EDGE_BLK
ACC_ROWS = 10240
ROWS_PER_TILE = ACC_ROWS // N_TILES
ZERO_BLK = 32


def _make_sc_agg(n_chunks):
    n_out = 2 if n_chunks == 1 else n_chunks
    mesh = plsc.VectorSubcoreMesh(core_axis_name="c", subcore_axis_name="s")

    @functools.partial(
        pl.kernel,
        out_type=jax.ShapeDtypeStruct((n_out, ACC_ROWS, SL, LN), jnp.bfloat16),
        mesh=mesh,
        compiler_params=pltpu.CompilerParams(use_tc_tiling_on_sc=False),
        scratch_types=[
            pltpu.VMEM_SHARED((ACC_ROWS, SL, LN), jnp.bfloat16),
            pltpu.VMEM((EDGE_BLK,), jnp.int32),
            pltpu.VMEM((EDGE_BLK,), jnp.int32),
            pltpu.VMEM((EDGE_BLK,), jnp.int32),
            pltpu.VMEM((EDGE_BLK,), jnp.int32),
            pltpu.VMEM((EDGE_BLK, SL, LN), jnp.bfloat16),
            pltpu.VMEM((EDGE_BLK, SL, LN), jnp.bfloat16),
            pltpu.VMEM((ZERO_BLK, SL, LN), jnp.bfloat16),
            pltpu.SemaphoreType.DMA,
            pltpu.SemaphoreType.DMA,
            pltpu.SemaphoreType.DMA,
            pltpu.SemaphoreType.DMA,
            pltpu.SemaphoreType.DMA,
            pltpu.SemaphoreType.DMA,
            pltpu.SemaphoreType.DMA,
        ],
    )
    def sc_agg(*refs):
        xs = refs[:n_chunks]
        src_hbm, dst_hbm, out_hbm = refs[n_chunks:n_chunks + 3]
        (agg_sh, s0, s1, d0, d1, rows0, rows1, zd_v,
         gsem0, gsem1, dsem0, dsem1, ssem0, ssem1, zsem) = refs[n_chunks + 3:]
        c = lax.axis_index("c")
        s = lax.axis_index("s")
        sbuf = (s0, s1)
        dbuf = (d0, d1)
        rbuf = (rows0, rows1)
        gsem = (gsem0, gsem1)
        dsem = (dsem0, dsem1)
        ssem = (ssem0, ssem1)

        def _zi(i, _):
            for t in range(SL):
                for k in range(LN // 32):
                    zd_v[i, t, pl.ds(k * 32, 32)] = jnp.zeros((32,), jnp.bfloat16)
            return 0
        lax.fori_loop(0, ZERO_BLK, _zi, 0)

        def one_pass(x_hbm, oi, blk0, nblk):
            nz = ROWS_PER_TILE // ZERO_BLK
            for j in range(nz):
                pltpu.async_copy(
                    zd_v,
                    agg_sh.at[pl.ds(s * ROWS_PER_TILE + j * ZERO_BLK, ZERO_BLK)],
                    zsem)
            for j in range(nz):
                pltpu.make_async_copy(
                    zd_v,
                    agg_sh.at[pl.ds(s * ROWS_PER_TILE + j * ZERO_BLK, ZERO_BLK)],
                    zsem).wait()
            plsc.subcore_barrier()

            def start_b(b, k):
                base = s * EDGES_PER_TILE + (blk0 + b) * EDGE_BLK
                pltpu.async_copy(dst_hbm.at[pl.ds(base, EDGE_BLK)], dbuf[k], dsem[k])
                pltpu.async_copy(src_hbm.at[pl.ds(base, EDGE_BLK)], sbuf[k], gsem[k])

            def fin_idx(b, k):
                base = s * EDGES_PER_TILE + (blk0 + b) * EDGE_BLK
                pltpu.make_async_copy(dst_hbm.at[pl.ds(base, EDGE_BLK)],
                                      dbuf[k], dsem[k]).wait()
                pltpu.make_async_copy(src_hbm.at[pl.ds(base, EDGE_BLK)],
                                      sbuf[k], gsem[k]).wait()
                pltpu.async_copy(x_hbm.at[sbuf[k]], rbuf[k], gsem[k])

            start_b(0, 0)
            fin_idx(0, 0)
            start_b(1, 1)
            fin_idx(1, 1)

            def pair(j, _):
                b0 = 2 * j
                pltpu.make_async_copy(x_hbm.at[sbuf[0]], rbuf[0], gsem[0]).wait()
                sc0 = pltpu.async_copy(rbuf[0], agg_sh.at[dbuf[0]], ssem[0],
                                       add=True)
                pltpu.make_async_copy(x_hbm.at[sbuf[1]], rbuf[1], gsem[1]).wait()
                sc1 = pltpu.async_copy(rbuf[1], agg_sh.at[dbuf[1]], ssem[1],
                                       add=True)
                sc0.wait()
                start_b(b0 + 2, 0)
                fin_idx(b0 + 2, 0)
                sc1.wait()
                start_b(b0 + 3, 1)
                fin_idx(b0 + 3, 1)
                return 0
            lax.fori_loop(0, nblk // 2 - 1, pair, 0)
            pltpu.make_async_copy(x_hbm.at[sbuf[0]], rbuf[0], gsem[0]).wait()
            pltpu.sync_copy(rbuf[0], agg_sh.at[dbuf[0]], add=True)
            pltpu.make_async_copy(x_hbm.at[sbuf[1]], rbuf[1], gsem[1]).wait()
            pltpu.sync_copy(rbuf[1], agg_sh.at[dbuf[1]], add=True)
            plsc.subcore_barrier()

            n_dr = ROWS_PER_TILE // EDGE_BLK
            out_desc = [None] * n_dr
            for j in range(n_dr):
                k = j % 2
                if j >= 2:
                    out_desc[j - 2].wait()
                row0 = s * ROWS_PER_TILE + j * EDGE_BLK
                pltpu.async_copy(agg_sh.at[pl.ds(row0, EDGE_BLK)],
                                 rbuf[k], gsem[k]).wait()
                out_desc[j] = pltpu.async_copy(
                    rbuf[k], out_hbm.at[oi, pl.ds(row0, EDGE_BLK)], ssem[k])
            out_desc[n_dr - 2].wait()
            out_desc[n_dr - 1].wait()

        if n_chunks == 1:
            for ccode in range(2):
                @pl.when(c == ccode)
                def _(ccode=ccode):
                    one_pass(xs[0], ccode, ccode * (N_BLKS // 2), N_BLKS // 2)
        else:
            for p in range(n_chunks // 2):
                for ccode in range(2):
                    @pl.when(c == ccode)
                    def _(p=p, ccode=ccode):
                        one_pass(xs[2 * p + ccode], 2 * p + ccode, 0, N_BLKS)

    return sc_agg


_sc_agg_1 = _make_sc_agg(1)
_sc_agg_4 = _make_sc_agg(4)


def _mlp1_body(x_ref, agg_ref, wa_ref, ba_ref, wb_ref, bb_ref, h_ref):
    agg = (agg_ref[0].astype(jnp.float32) + agg_ref[1].astype(jnp.float32))
    xin = (x_ref[...] + agg).astype(jnp.bfloat16)
    t = jnp.dot(xin, wa_ref[...], preferred_element_type=jnp.float32) + ba_ref[...]
    t = jnp.maximum(t, 0.0).astype(jnp.bfloat16)
    h = jnp.dot(t, wb_ref[...], preferred_element_type=jnp.float32) + bb_ref[...]
    h_ref[...] = jnp.maximum(h, 0.0).astype(jnp.bfloat16)


def _mlp2_body(h_ref, agg_ref, wa_ref, ba_ref, wb_ref, bb_ref, o_ref):
    i = pl.program_id(0)
    zin = h_ref[...] + jnp.concatenate(
        [agg_ref[j] for j in range(HID_F // CHUNK_W)], axis=-1)
    t = jnp.dot(zin, wa_ref[...], preferred_element_type=jnp.float32) + ba_ref[...]
    t = jnp.maximum(t, 0.0).astype(jnp.bfloat16)
    r = jnp.dot(t, wb_ref[...], preferred_element_type=jnp.float32) + bb_ref[...]
    r = jnp.maximum(r, 0.0)
    part = jnp.sum(r, axis=0, keepdims=True)

    @pl.when(i == 0)
    def _():
        o_ref[...] = part

    @pl.when(i != 0)
    def _():
        o_ref[...] = o_ref[...] + part


ROW_BLK = 1000
N_ROW_BLKS = N_NODES // ROW_BLK


def _mlp1(x, agg1, W1a, b1a, W1b, b1b):
    return pl.pallas_call(
        _mlp1_body,
        grid=(N_ROW_BLKS,),
        in_specs=[
            pl.BlockSpec((ROW_BLK, IN_F), lambda i: (i, 0)),
            pl.BlockSpec((2, ROW_BLK, CHUNK_W), lambda i: (0, i, 0)),
            pl.BlockSpec((IN_F, HID_F), lambda i: (0, 0)),
            pl.BlockSpec((1, HID_F), lambda i: (0, 0)),
            pl.BlockSpec((HID_F, HID_F), lambda i: (0, 0)),
            pl.BlockSpec((1, HID_F), lambda i: (0, 0)),
        ],
        out_specs=pl.BlockSpec((ROW_BLK, HID_F), lambda i: (i, 0)),
        out_shape=jax.ShapeDtypeStruct((N_NODES, HID_F), jnp.bfloat16),
    )(x, agg1, W1a.astype(jnp.bfloat16), b1a.reshape(1, -1),
      W1b.astype(jnp.bfloat16), b1b.reshape(1, -1))


def _mlp2(h, agg2, W2a, b2a, W2b, b2b):
    out = pl.pallas_call(
        _mlp2_body,
        grid=(N_ROW_BLKS,),
        in_specs=[
            pl.BlockSpec((ROW_BLK, HID_F), lambda i: (i, 0)),
            pl.BlockSpec((HID_F // CHUNK_W, ROW_BLK, CHUNK_W), lambda i: (0, i, 0)),
            pl.BlockSpec((HID_F, HID_F), lambda i: (0, 0)),
            pl.BlockSpec((1, HID_F), lambda i: (0, 0)),
            pl.BlockSpec((HID_F, IN_F), lambda i: (0, 0)),
            pl.BlockSpec((1, IN_F), lambda i: (0, 0)),
        ],
        out_specs=pl.BlockSpec((1, IN_F), lambda i: (0, 0)),
        out_shape=jax.ShapeDtypeStruct((1, IN_F), jnp.float32),
    )(h, agg2, W2a.astype(jnp.bfloat16), b2a.reshape(1, -1),
      W2b.astype(jnp.bfloat16), b2b.reshape(1, -1))
    return out.reshape(IN_F)


def kernel(x, edge_index, W1a, b1a, W1b, b1b, W2a, b2a, W2b, b2b):
    e = edge_index.astype(jnp.int32)
    pad = EDGES_PER_TILE - N_EDGES // N_TILES
    src = jnp.pad(e[0].reshape(N_TILES, -1), ((0, 0), (0, pad)),
                  constant_values=0).reshape(-1)
    dst = jnp.pad(e[1].reshape(N_TILES, -1), ((0, 0), (0, pad)),
                  constant_values=ACC_ROWS - 1).reshape(-1)

    xb = x.astype(jnp.bfloat16).reshape(N_NODES, SL, LN)
    agg1 = _sc_agg_1(xb, src, dst).reshape(2, ACC_ROWS, CHUNK_W)

    h = _mlp1(x, agg1, W1a, b1a, W1b, b1b)

    h_chunks = tuple(
        h[:, i * CHUNK_W:(i + 1) * CHUNK_W].reshape(N_NODES, SL, LN)
        for i in range(HID_F // CHUNK_W))
    agg2 = _sc_agg_4(*h_chunks, src, dst).reshape(4, ACC_ROWS, CHUNK_W)

    return _mlp2(h, agg2, W2a, b2a, W2b, b2b)
